# baseline (speedup 1.0000x reference)
import functools

import jax
import jax.numpy as jnp
from jax import lax
from jax.experimental import pallas as pl
from jax.experimental.pallas import tpu as pltpu
from jax.experimental.pallas import tpu_sc as plsc

N = 10000
E = 320000
IN_DIM = 128
HID = 64
HHID = HID // 2

NC = 2
NS = 16
NW = NC * NS
CHUNK = 128
NCH = E // CHUNK
DG_LO = NCH // NW
DG_XTRA = NCH - DG_LO * NW
AG_LO = NCH // NS
AG_XTRA = NCH - AG_LO * NS
AGG_ROWS = 10240
ZROWS = AGG_ROWS // NS
NBUF = 8
YST = 624

_mesh = plsc.VectorSubcoreMesh(core_axis_name="c", subcore_axis_name="s")
_sc_params = pltpu.CompilerParams(use_tc_tiling_on_sc=False)


def _sc_degree(ei3, zeros16, ones16):
    @functools.partial(
        pl.kernel,
        out_type=jax.ShapeDtypeStruct((NC, AGG_ROWS, 16), jnp.float32),
        mesh=_mesh,
        compiler_params=_sc_params,
        scratch_types=[
            pltpu.VMEM((DG_LO + 1, CHUNK), jnp.int32),
            pltpu.VMEM((CHUNK, 16), jnp.float32),
            pltpu.VMEM_SHARED((AGG_ROWS, 16), jnp.float32),
            pltpu.SemaphoreType.DMA((NBUF,)),
        ],
    )
    def deg_kernel(ei_hbm, z_hbm, ones_hbm, out_hbm, idx_v, ones_v, table, ssem):
        core = lax.axis_index("c")
        sub = lax.axis_index("s")
        wid = sub * NC + core
        start = wid * DG_LO + jnp.minimum(wid, DG_XTRA)
        pltpu.sync_copy(z_hbm.at[pl.ds(sub * ZROWS, ZROWS)],
                        table.at[pl.ds(sub * ZROWS, ZROWS)])
        pltpu.sync_copy(ones_hbm, ones_v)

        @pl.when(wid < DG_XTRA)
        def _():
            pltpu.sync_copy(ei_hbm.at[1, pl.ds(start, DG_LO + 1)],
                            idx_v.at[pl.ds(0, DG_LO + 1)])

        @pl.when(wid >= DG_XTRA)
        def _():
            pltpu.sync_copy(ei_hbm.at[1, pl.ds(start, DG_LO)],
                            idx_v.at[pl.ds(0, DG_LO)])

        plsc.subcore_barrier()

        def scatter_chunks(cnt):
            ngrp, tail = divmod(cnt, NBUF)

            @pl.loop(0, ngrp)
            def _(t):
                c0 = t * NBUF
                descs = []
                for j in range(NBUF):
                    descs.append(pltpu.async_copy(
                        ones_v, table.at[idx_v.at[c0 + j]], ssem.at[j],
                        add=True))
                for d in descs:
                    d.wait()

            descs = []
            for j in range(tail):
                descs.append(pltpu.async_copy(
                    ones_v, table.at[idx_v.at[ngrp * NBUF + j]], ssem.at[j],
                    add=True))
            for d in descs:
                d.wait()

        @pl.when(wid < DG_XTRA)
        def _():
            scatter_chunks(DG_LO + 1)

        @pl.when(wid >= DG_XTRA)
        def _():
            scatter_chunks(DG_LO)

        plsc.subcore_barrier()
        pltpu.sync_copy(table.at[pl.ds(sub * ZROWS, ZROWS)],
                        out_hbm.at[core, pl.ds(sub * ZROWS, ZROWS)])

    return deg_kernel(ei3, zeros16, ones16)


def _sc_aggregate(y, ei3, zeros32):
    @functools.partial(
        pl.kernel,
        out_type=jax.ShapeDtypeStruct((AGG_ROWS, HID), jnp.float32),
        mesh=_mesh,
        compiler_params=_sc_params,
        scratch_types=[
            pltpu.VMEM((AG_LO + 1, CHUNK), jnp.int32),
            pltpu.VMEM((AG_LO + 1, CHUNK), jnp.int32),
            pltpu.VMEM((NBUF, CHUNK, HHID), jnp.float32),
            pltpu.VMEM_SHARED((AGG_ROWS, HHID), jnp.float32),
            pltpu.VMEM_SHARED((N, HHID), jnp.float32),
            pltpu.SemaphoreType.DMA((NBUF,)),
            pltpu.SemaphoreType.DMA((NBUF,)),
        ],
    )
    def agg_kernel(y_hbm, ei_hbm, z_hbm, out_hbm,
                   si_v, di_v, rows_v, table, ytab, gsem, ssem):
        core = lax.axis_index("c")
        sub = lax.axis_index("s")
        start = sub * AG_LO + jnp.minimum(sub, AG_XTRA)
        pltpu.sync_copy(z_hbm.at[pl.ds(sub * ZROWS, ZROWS)],
                        table.at[pl.ds(sub * ZROWS, ZROWS)])

        @pl.when(sub < NS - 1)
        def _():
            pltpu.sync_copy(
                y_hbm.at[pl.ds(sub * YST, YST), pl.ds(core * HHID, HHID)],
                ytab.at[pl.ds(sub * YST, YST)])

        @pl.when(sub == NS - 1)
        def _():
            pltpu.sync_copy(
                y_hbm.at[pl.ds((NS - 1) * YST, N - (NS - 1) * YST),
                         pl.ds(core * HHID, HHID)],
                ytab.at[pl.ds((NS - 1) * YST, N - (NS - 1) * YST)])

        @pl.when(sub < AG_XTRA)
        def _():
            pltpu.sync_copy(ei_hbm.at[0, pl.ds(start, AG_LO + 1)],
                            si_v.at[pl.ds(0, AG_LO + 1)])
            pltpu.sync_copy(ei_hbm.at[1, pl.ds(start, AG_LO + 1)],
                            di_v.at[pl.ds(0, AG_LO + 1)])

        @pl.when(sub >= AG_XTRA)
        def _():
            pltpu.sync_copy(ei_hbm.at[0, pl.ds(start, AG_LO)],
                            si_v.at[pl.ds(0, AG_LO)])
            pltpu.sync_copy(ei_hbm.at[1, pl.ds(start, AG_LO)],
                            di_v.at[pl.ds(0, AG_LO)])

        plsc.subcore_barrier()

        def move_chunks(cnt):
            ngrp, tail = divmod(cnt, NBUF)

            @pl.loop(0, ngrp)
            def _(t):
                c0 = t * NBUF
                gds = []
                for j in range(NBUF):
                    gds.append(pltpu.async_copy(
                        ytab.at[si_v.at[c0 + j]], rows_v.at[j], gsem.at[j]))
                sds = []
                for j in range(NBUF):
                    gds[j].wait()
                    sds.append(pltpu.async_copy(
                        rows_v.at[j], table.at[di_v.at[c0 + j]], ssem.at[j],
                        add=True))
                for d in sds:
                    d.wait()

            c0 = ngrp * NBUF
            gds = []
            for j in range(tail):
                gds.append(pltpu.async_copy(
                    ytab.at[si_v.at[c0 + j]], rows_v.at[j], gsem.at[j]))
            sds = []
            for j in range(tail):
                gds[j].wait()
                sds.append(pltpu.async_copy(
                    rows_v.at[j], table.at[di_v.at[c0 + j]], ssem.at[j],
                    add=True))
            for d in sds:
                d.wait()

        @pl.when(sub < AG_XTRA)
        def _():
            move_chunks(AG_LO + 1)

        @pl.when(sub >= AG_XTRA)
        def _():
            move_chunks(AG_LO)

        plsc.subcore_barrier()
        pltpu.sync_copy(table.at[pl.ds(sub * ZROWS, ZROWS)],
                        out_hbm.at[pl.ds(sub * ZROWS, ZROWS),
                                   pl.ds(core * HHID, HHID)])

    return agg_kernel(y, ei3, zeros32)


_B = 1000


def _mm1_body(x_ref, w_ref, o_ref):
    o_ref[...] = jnp.dot(x_ref[...], w_ref[...],
                         preferred_element_type=jnp.float32)


def _tc_mm1(x, W1):
    return pl.pallas_call(
        _mm1_body,
        grid=(N // _B,),
        in_specs=[pl.BlockSpec((_B, IN_DIM), lambda i: (i, 0)),
                  pl.BlockSpec((IN_DIM, HID), lambda i: (0, 0))],
        out_specs=pl.BlockSpec((_B, HID), lambda i: (i, 0)),
        out_shape=jax.ShapeDtypeStruct((N, HID), jnp.float32),
    )(x, W1)


def _scale_body(dp_ref, xw_ref, y_ref, dinv_ref):
    deg = dp_ref[0] + dp_ref[1] + 1.0
    dinv = lax.rsqrt(deg)
    dinv_ref[...] = dinv
    y_ref[...] = xw_ref[...] * dinv[:, :1]


def _tc_scale(degp, xw):
    return pl.pallas_call(
        _scale_body,
        grid=(N // _B,),
        in_specs=[pl.BlockSpec((NC, _B, 16), lambda i: (0, i, 0)),
                  pl.BlockSpec((_B, HID), lambda i: (i, 0))],
        out_specs=[pl.BlockSpec((_B, HID), lambda i: (i, 0)),
                   pl.BlockSpec((_B, 16), lambda i: (i, 0))],
        out_shape=[jax.ShapeDtypeStruct((N, HID), jnp.float32),
                   jax.ShapeDtypeStruct((N, 16), jnp.float32)],
    )(degp, xw)


def _mid_body(dinv_ref, y_ref, a_ref, b1_ref, w2_ref, o_ref):
    dinv1 = dinv_ref[...][:, :1]
    h = jax.nn.relu(dinv1 * (y_ref[...] + a_ref[...]) + b1_ref[...])
    o_ref[...] = jnp.dot(h, w2_ref[...],
                         preferred_element_type=jnp.float32) * dinv1


def _tc_mid(dinv, y1, agg1, b1, W2):
    return pl.pallas_call(
        _mid_body,
        grid=(N // _B,),
        in_specs=[pl.BlockSpec((_B, 16), lambda i: (i, 0)),
                  pl.BlockSpec((_B, HID), lambda i: (i, 0)),
                  pl.BlockSpec((_B, HID), lambda i: (i, 0)),
                  pl.BlockSpec((1, HID), lambda i: (0, 0)),
                  pl.BlockSpec((HID, HID), lambda i: (0, 0))],
        out_specs=pl.BlockSpec((_B, HID), lambda i: (i, 0)),
        out_shape=jax.ShapeDtypeStruct((N, HID), jnp.float32),
    )(dinv, y1, agg1, b1, W2)


def _fin_body(dinv_ref, y_ref, c_ref, b2_ref, wfc_ref, bfc_ref, o_ref):
    dinv1 = dinv_ref[...][:, :1]
    h = jax.nn.relu(dinv1 * (y_ref[...] + c_ref[...]) + b2_ref[...])
    z = jnp.dot(h, wfc_ref[...], preferred_element_type=jnp.float32)
    o_ref[...] = jax.nn.sigmoid(z + bfc_ref[...])


def _tc_final(dinv, y2, agg2, b2, Wfc, bfc):
    return pl.pallas_call(
        _fin_body,
        grid=(N // _B,),
        in_specs=[pl.BlockSpec((_B, 16), lambda i: (i, 0)),
                  pl.BlockSpec((_B, HID), lambda i: (i, 0)),
                  pl.BlockSpec((_B, HID), lambda i: (i, 0)),
                  pl.BlockSpec((1, HID), lambda i: (0, 0)),
                  pl.BlockSpec((HID, 1), lambda i: (0, 0)),
                  pl.BlockSpec((1, 1), lambda i: (0, 0))],
        out_specs=pl.BlockSpec((_B, 1), lambda i: (i, 0)),
        out_shape=jax.ShapeDtypeStruct((N, 1), jnp.float32),
    )(dinv, y2, agg2, b2, Wfc, bfc)


def kernel(x, edge_index, W1, b1, W2, b2, Wfc, bfc):
    ei3 = edge_index.astype(jnp.int32).reshape(2, NCH, CHUNK)
    zeros32 = jnp.zeros((AGG_ROWS, HHID), jnp.float32)
    zeros16 = jnp.zeros((AGG_ROWS, 16), jnp.float32)
    ones16 = jnp.ones((CHUNK, 16), jnp.float32)
    b1r = b1.reshape(1, HID)
    b2r = b2.reshape(1, HID)
    bfcr = bfc.reshape(1, 1)

    degp = _sc_degree(ei3, zeros16, ones16)
    xw1 = _tc_mm1(x, W1)
    y1, dinv = _tc_scale(degp, xw1)
    agg1 = _sc_aggregate(y1, ei3, zeros32)
    y2 = _tc_mid(dinv, y1, agg1, b1r, W2)
    agg2 = _sc_aggregate(y2, ei3, zeros32)
    return _tc_final(dinv, y2, agg2, b2r, Wfc, bfcr)


# speedup vs baseline: 1.0477x; 1.0477x over previous
import functools

import jax
import jax.numpy as jnp
from jax import lax
from jax.experimental import pallas as pl
from jax.experimental.pallas import tpu as pltpu
from jax.experimental.pallas import tpu_sc as plsc

N = 10000
E = 320000
IN_DIM = 128
HID = 64
HHID = HID // 2

NC = 2
NS = 16
NW = NC * NS
CHUNK = 128
NCH = E // CHUNK
DG_LO = NCH // NW
DG_XTRA = NCH - DG_LO * NW
AG_LO = NCH // NS
AG_XTRA = NCH - AG_LO * NS
AGG_ROWS = 10240
ZROWS = AGG_ROWS // NS
NBUF = 8
YST = 624

_mesh = plsc.VectorSubcoreMesh(core_axis_name="c", subcore_axis_name="s")
_sc_params = pltpu.CompilerParams(use_tc_tiling_on_sc=False)


def _sc_degree(ei3, zeros16, ones16):
    @functools.partial(
        pl.kernel,
        out_type=jax.ShapeDtypeStruct((NC, AGG_ROWS, 16), jnp.float32),
        mesh=_mesh,
        compiler_params=_sc_params,
        scratch_types=[
            pltpu.VMEM((DG_LO + 1, CHUNK), jnp.int32),
            pltpu.VMEM((CHUNK, 16), jnp.float32),
            pltpu.VMEM_SHARED((AGG_ROWS, 16), jnp.float32),
            pltpu.SemaphoreType.DMA((NBUF,)),
        ],
    )
    def deg_kernel(ei_hbm, z_hbm, ones_hbm, out_hbm, idx_v, ones_v, table, ssem):
        core = lax.axis_index("c")
        sub = lax.axis_index("s")
        wid = sub * NC + core
        start = wid * DG_LO + jnp.minimum(wid, DG_XTRA)
        pltpu.sync_copy(z_hbm.at[pl.ds(sub * ZROWS, ZROWS)],
                        table.at[pl.ds(sub * ZROWS, ZROWS)])
        pltpu.sync_copy(ones_hbm, ones_v)

        @pl.when(wid < DG_XTRA)
        def _():
            pltpu.sync_copy(ei_hbm.at[1, pl.ds(start, DG_LO + 1)],
                            idx_v.at[pl.ds(0, DG_LO + 1)])

        @pl.when(wid >= DG_XTRA)
        def _():
            pltpu.sync_copy(ei_hbm.at[1, pl.ds(start, DG_LO)],
                            idx_v.at[pl.ds(0, DG_LO)])

        plsc.subcore_barrier()

        def scatter_chunks(cnt):
            ngrp, tail = divmod(cnt, NBUF)

            @pl.loop(0, ngrp)
            def _(t):
                c0 = t * NBUF
                descs = []
                for j in range(NBUF):
                    descs.append(pltpu.async_copy(
                        ones_v, table.at[idx_v.at[c0 + j]], ssem.at[j],
                        add=True))
                for d in descs:
                    d.wait()

            descs = []
            for j in range(tail):
                descs.append(pltpu.async_copy(
                    ones_v, table.at[idx_v.at[ngrp * NBUF + j]], ssem.at[j],
                    add=True))
            for d in descs:
                d.wait()

        @pl.when(wid < DG_XTRA)
        def _():
            scatter_chunks(DG_LO + 1)

        @pl.when(wid >= DG_XTRA)
        def _():
            scatter_chunks(DG_LO)

        plsc.subcore_barrier()
        pltpu.sync_copy(table.at[pl.ds(sub * ZROWS, ZROWS)],
                        out_hbm.at[core, pl.ds(sub * ZROWS, ZROWS)])

    return deg_kernel(ei3, zeros16, ones16)


def _sc_aggregate(y, ei3, zeros32):
    @functools.partial(
        pl.kernel,
        out_type=jax.ShapeDtypeStruct((AGG_ROWS, HID), jnp.float32),
        mesh=_mesh,
        compiler_params=_sc_params,
        scratch_types=[
            pltpu.VMEM((AG_LO + 1, CHUNK), jnp.int32),
            pltpu.VMEM((AG_LO + 1, CHUNK), jnp.int32),
            pltpu.VMEM((NBUF, CHUNK, HHID), jnp.float32),
            pltpu.VMEM_SHARED((AGG_ROWS, HHID), jnp.float32),
            pltpu.VMEM_SHARED((N, HHID), jnp.float32),
            pltpu.SemaphoreType.DMA((NBUF,)),
            pltpu.SemaphoreType.DMA((NBUF,)),
        ],
    )
    def agg_kernel(y_hbm, ei_hbm, z_hbm, out_hbm,
                   si_v, di_v, rows_v, table, ytab, gsem, ssem):
        core = lax.axis_index("c")
        sub = lax.axis_index("s")
        start = sub * AG_LO + jnp.minimum(sub, AG_XTRA)
        pltpu.sync_copy(z_hbm.at[pl.ds(sub * ZROWS, ZROWS)],
                        table.at[pl.ds(sub * ZROWS, ZROWS)])

        @pl.when(sub < NS - 1)
        def _():
            pltpu.sync_copy(
                y_hbm.at[pl.ds(sub * YST, YST), pl.ds(core * HHID, HHID)],
                ytab.at[pl.ds(sub * YST, YST)])

        @pl.when(sub == NS - 1)
        def _():
            pltpu.sync_copy(
                y_hbm.at[pl.ds((NS - 1) * YST, N - (NS - 1) * YST),
                         pl.ds(core * HHID, HHID)],
                ytab.at[pl.ds((NS - 1) * YST, N - (NS - 1) * YST)])

        @pl.when(sub < AG_XTRA)
        def _():
            pltpu.sync_copy(ei_hbm.at[0, pl.ds(start, AG_LO + 1)],
                            si_v.at[pl.ds(0, AG_LO + 1)])
            pltpu.sync_copy(ei_hbm.at[1, pl.ds(start, AG_LO + 1)],
                            di_v.at[pl.ds(0, AG_LO + 1)])

        @pl.when(sub >= AG_XTRA)
        def _():
            pltpu.sync_copy(ei_hbm.at[0, pl.ds(start, AG_LO)],
                            si_v.at[pl.ds(0, AG_LO)])
            pltpu.sync_copy(ei_hbm.at[1, pl.ds(start, AG_LO)],
                            di_v.at[pl.ds(0, AG_LO)])

        plsc.subcore_barrier()

        def move_chunks(cnt):
            ngrp, tail = divmod(cnt, NBUF)

            @pl.loop(0, ngrp)
            def _(t):
                c0 = t * NBUF
                gds = []
                for j in range(NBUF):
                    gds.append(pltpu.async_copy(
                        ytab.at[si_v.at[c0 + j]], rows_v.at[j], gsem.at[j]))
                sds = []
                for j in range(NBUF):
                    gds[j].wait()
                    sds.append(pltpu.async_copy(
                        rows_v.at[j], table.at[di_v.at[c0 + j]], ssem.at[j],
                        add=True))
                for d in sds:
                    d.wait()

            c0 = ngrp * NBUF
            gds = []
            for j in range(tail):
                gds.append(pltpu.async_copy(
                    ytab.at[si_v.at[c0 + j]], rows_v.at[j], gsem.at[j]))
            sds = []
            for j in range(tail):
                gds[j].wait()
                sds.append(pltpu.async_copy(
                    rows_v.at[j], table.at[di_v.at[c0 + j]], ssem.at[j],
                    add=True))
            for d in sds:
                d.wait()

        @pl.when(sub < AG_XTRA)
        def _():
            move_chunks(AG_LO + 1)

        @pl.when(sub >= AG_XTRA)
        def _():
            move_chunks(AG_LO)

        plsc.subcore_barrier()
        pltpu.sync_copy(table.at[pl.ds(sub * ZROWS, ZROWS)],
                        out_hbm.at[pl.ds(sub * ZROWS, ZROWS),
                                   pl.ds(core * HHID, HHID)])

    return agg_kernel(y, ei3, zeros32)


_B = 5000


def _mm1_body(x_ref, w_ref, o_ref):
    o_ref[...] = jnp.dot(x_ref[...], w_ref[...],
                         preferred_element_type=jnp.float32)


def _tc_mm1(x, W1):
    return pl.pallas_call(
        _mm1_body,
        grid=(N // _B,),
        in_specs=[pl.BlockSpec((_B, IN_DIM), lambda i: (i, 0)),
                  pl.BlockSpec((IN_DIM, HID), lambda i: (0, 0))],
        out_specs=pl.BlockSpec((_B, HID), lambda i: (i, 0)),
        out_shape=jax.ShapeDtypeStruct((N, HID), jnp.float32),
    )(x, W1)


def _scale_body(dp_ref, xw_ref, y_ref, dinv_ref):
    deg = dp_ref[0] + dp_ref[1] + 1.0
    dinv = lax.rsqrt(deg)
    dinv_ref[...] = dinv
    y_ref[...] = xw_ref[...] * dinv[:, :1]


def _tc_scale(degp, xw):
    return pl.pallas_call(
        _scale_body,
        grid=(N // _B,),
        in_specs=[pl.BlockSpec((NC, _B, 16), lambda i: (0, i, 0)),
                  pl.BlockSpec((_B, HID), lambda i: (i, 0))],
        out_specs=[pl.BlockSpec((_B, HID), lambda i: (i, 0)),
                   pl.BlockSpec((_B, 16), lambda i: (i, 0))],
        out_shape=[jax.ShapeDtypeStruct((N, HID), jnp.float32),
                   jax.ShapeDtypeStruct((N, 16), jnp.float32)],
    )(degp, xw)


def _mid_body(dinv_ref, y_ref, a_ref, b1_ref, w2_ref, o_ref):
    dinv1 = dinv_ref[...][:, :1]
    h = jax.nn.relu(dinv1 * (y_ref[...] + a_ref[...]) + b1_ref[...])
    o_ref[...] = jnp.dot(h, w2_ref[...],
                         preferred_element_type=jnp.float32) * dinv1


def _tc_mid(dinv, y1, agg1, b1, W2):
    return pl.pallas_call(
        _mid_body,
        grid=(N // _B,),
        in_specs=[pl.BlockSpec((_B, 16), lambda i: (i, 0)),
                  pl.BlockSpec((_B, HID), lambda i: (i, 0)),
                  pl.BlockSpec((_B, HID), lambda i: (i, 0)),
                  pl.BlockSpec((1, HID), lambda i: (0, 0)),
                  pl.BlockSpec((HID, HID), lambda i: (0, 0))],
        out_specs=pl.BlockSpec((_B, HID), lambda i: (i, 0)),
        out_shape=jax.ShapeDtypeStruct((N, HID), jnp.float32),
    )(dinv, y1, agg1, b1, W2)


def _fin_body(dinv_ref, y_ref, c_ref, b2_ref, wfc_ref, bfc_ref, o_ref):
    dinv1 = dinv_ref[...][:, :1]
    h = jax.nn.relu(dinv1 * (y_ref[...] + c_ref[...]) + b2_ref[...])
    z = jnp.dot(h, wfc_ref[...], preferred_element_type=jnp.float32)
    o_ref[...] = jax.nn.sigmoid(z + bfc_ref[...])


def _tc_final(dinv, y2, agg2, b2, Wfc, bfc):
    return pl.pallas_call(
        _fin_body,
        grid=(N // _B,),
        in_specs=[pl.BlockSpec((_B, 16), lambda i: (i, 0)),
                  pl.BlockSpec((_B, HID), lambda i: (i, 0)),
                  pl.BlockSpec((_B, HID), lambda i: (i, 0)),
                  pl.BlockSpec((1, HID), lambda i: (0, 0)),
                  pl.BlockSpec((HID, 1), lambda i: (0, 0)),
                  pl.BlockSpec((1, 1), lambda i: (0, 0))],
        out_specs=pl.BlockSpec((_B, 1), lambda i: (i, 0)),
        out_shape=jax.ShapeDtypeStruct((N, 1), jnp.float32),
    )(dinv, y2, agg2, b2, Wfc, bfc)


def kernel(x, edge_index, W1, b1, W2, b2, Wfc, bfc):
    ei3 = edge_index.astype(jnp.int32).reshape(2, NCH, CHUNK)
    zeros32 = jnp.zeros((AGG_ROWS, HHID), jnp.float32)
    zeros16 = jnp.zeros((AGG_ROWS, 16), jnp.float32)
    ones16 = jnp.ones((CHUNK, 16), jnp.float32)
    b1r = b1.reshape(1, HID)
    b2r = b2.reshape(1, HID)
    bfcr = bfc.reshape(1, 1)

    degp = _sc_degree(ei3, zeros16, ones16)
    xw1 = _tc_mm1(x, W1)
    y1, dinv = _tc_scale(degp, xw1)
    agg1 = _sc_aggregate(y1, ei3, zeros32)
    y2 = _tc_mid(dinv, y1, agg1, b1r, W2)
    agg2 = _sc_aggregate(y2, ei3, zeros32)
    return _tc_final(dinv, y2, agg2, b2r, Wfc, bfcr)
